# baseline (speedup 1.0000x reference)
import functools

import jax
import jax.numpy as jnp
from jax import lax
from jax.experimental import pallas as pl
from jax.experimental.pallas import tpu as pltpu
from jax.experimental.pallas import tpu_sc as plsc

NUM_FIELDS = 26
NUM_FEATURES = 100000
LATENT_DIM = 16
BATCH = 4096

NUM_WORKERS = 32
CHUNK_FEAT = 128
CHUNKS_PER_FIELD = -(-NUM_FEATURES // CHUNK_FEAT)
TAIL_FEAT = NUM_FEATURES - (CHUNKS_PER_FIELD - 1) * CHUNK_FEAT
TOTAL_CHUNKS = NUM_FIELDS * CHUNKS_PER_FIELD
SLAB_W = CHUNK_FEAT + 1
F_ROWS = CHUNKS_PER_FIELD * 16
NUM_UNITS = NUM_FIELDS * F_ROWS

SAMPLES_PER_W = BATCH // NUM_WORKERS
PAIRS_PER_W = SAMPLES_PER_W * NUM_FIELDS
CHUNK_S = 16
CHUNK_P = CHUNK_S * NUM_FIELDS
N_CHUNKS = SAMPLES_PER_W // CHUNK_S
IDX_DMA = 104


@functools.partial(
    pl.kernel,
    out_type=jax.ShapeDtypeStruct((NUM_UNITS, 128), jnp.float32),
    mesh=plsc.VectorSubcoreMesh(core_axis_name="c", subcore_axis_name="s"),
    compiler_params=pltpu.CompilerParams(needs_layout_passes=False),
    scratch_types=[
        pltpu.VMEM((16, SLAB_W), jnp.float32),
        pltpu.VMEM((16, SLAB_W), jnp.float32),
        pltpu.VMEM((16, 128), jnp.float32),
        pltpu.VMEM((16, 128), jnp.float32),
        pltpu.SemaphoreType.DMA,
        pltpu.SemaphoreType.DMA,
        pltpu.SemaphoreType.DMA,
        pltpu.SemaphoreType.DMA,
    ],
)
def _pack_sc(embt_hbm, tail_hbm, packed_hbm, sl_a, sl_b, ob_a, ob_b,
             sra, srb, swa, swb):
    wid = lax.axis_index("s") * 2 + lax.axis_index("c")
    lo = wid * TOTAL_CHUNKS // NUM_WORKERS
    hi = (wid + 1) * TOTAL_CHUNKS // NUM_WORKERS

    lanes = lax.iota(jnp.int32, 16)

    def fetch(cid, slab, sem):
        @pl.when(cid < hi)
        def _():
            i = cid // CHUNKS_PER_FIELD
            cb = cid % CHUNKS_PER_FIELD

            @pl.when(cb < CHUNKS_PER_FIELD - 1)
            def _():
                pltpu.async_copy(
                    embt_hbm.at[i, :, pl.ds(cb * CHUNK_FEAT, CHUNK_FEAT)],
                    slab.at[:, pl.ds(0, CHUNK_FEAT)],
                    sem,
                )

    def wait_fetch(cid, slab, sem):
        @pl.when((cid < hi) & (cid % CHUNKS_PER_FIELD < CHUNKS_PER_FIELD - 1))
        def _():
            pltpu.make_async_copy(embt_hbm.at[0, :, pl.ds(0, CHUNK_FEAT)],
                                  slab.at[:, pl.ds(0, CHUNK_FEAT)], sem).wait()

    def process(cid, slab, outb, semw):
        @pl.when(cid < hi)
        def _():
            i = cid // CHUNKS_PER_FIELD
            cb = cid % CHUNKS_PER_FIELD

            @pl.when(cid >= lo + 2)
            def _():
                pltpu.make_async_copy(
                    outb, packed_hbm.at[pl.ds(0, 16), :], semw
                ).wait()

            @pl.when(cb < CHUNKS_PER_FIELD - 1)
            def _():
                for r in range(16):
                    for j in range(8):
                        col = jnp.full((16,), r * 8 + j, jnp.int32)
                        v = plsc.load_gather(slab, [lanes, col])
                        outb[r, pl.ds(j * 16, 16)] = v

            @pl.when(cb == CHUNKS_PER_FIELD - 1)
            def _():
                pltpu.sync_copy(tail_hbm.at[i], outb.at[pl.ds(0, 8), :])

            pltpu.async_copy(
                outb,
                packed_hbm.at[pl.ds(i * F_ROWS + cb * 16, 16), :],
                semw,
            )

    fetch(lo, sl_a, sra)
    n = hi - lo

    def body(t, _):
        ce = lo + 2 * t
        fetch(ce + 1, sl_b, srb)
        wait_fetch(ce, sl_a, sra)
        process(ce, sl_a, ob_a, swa)
        fetch(ce + 2, sl_a, sra)
        wait_fetch(ce + 1, sl_b, srb)
        process(ce + 1, sl_b, ob_b, swb)
        return _

    lax.fori_loop(0, (n + 1) // 2, body, 0)
    pltpu.make_async_copy(ob_a, packed_hbm.at[pl.ds(0, 16), :], swa).wait()
    pltpu.make_async_copy(ob_b, packed_hbm.at[pl.ds(0, 16), :], swb).wait()


@functools.partial(
    pl.kernel,
    out_type=jax.ShapeDtypeStruct((BATCH,), jnp.float32),
    mesh=plsc.VectorSubcoreMesh(core_axis_name="c", subcore_axis_name="s"),
    compiler_params=pltpu.CompilerParams(needs_layout_passes=False),
    scratch_types=[
        pltpu.VMEM((PAIRS_PER_W,), jnp.int32),
        pltpu.VMEM((PAIRS_PER_W,), jnp.int32),
        pltpu.VMEM((2, CHUNK_P, 128), jnp.float32),
        pltpu.VMEM((SAMPLES_PER_W,), jnp.float32),
        pltpu.SemaphoreType.DMA,
        pltpu.SemaphoreType.DMA,
    ],
)
def _ffm_sc(table_hbm, x_hbm, u_hbm, out_hbm, xv, uv, buf, out_v, sem_a, sem_b):
    wid = lax.axis_index("s") * 2 + lax.axis_index("c")
    base = wid * PAIRS_PER_W

    pltpu.sync_copy(x_hbm.at[pl.ds(base, PAIRS_PER_W)], xv)
    pltpu.sync_copy(u_hbm.at[pl.ds(base, PAIRS_PER_W)], uv)

    lanes = lax.iota(jnp.int32, 16)
    zero_f = jnp.zeros((16,), jnp.float32)

    def fire(k, buf_ref, sem):
        for q in range(CHUNK_P // IDX_DMA):
            pltpu.async_copy(
                table_hbm.at[uv.at[pl.ds(k * CHUNK_P + q * IDX_DMA, IDX_DMA)]],
                buf_ref.at[pl.ds(q * IDX_DMA, IDX_DMA), :],
                sem,
            )

    def drain(buf_ref, sem):
        pltpu.make_async_copy(
            table_hbm.at[pl.ds(0, CHUNK_P), :], buf_ref, sem
        ).wait()

    def consume(k, buf_ref):
        p0 = k * CHUNK_P

        def field_body(i, carry):
            q, lin = carry[0], carry[1]
            accs = carry[2:]
            pv = p0 + lanes * NUM_FIELDS + i
            xr = plsc.load_gather(xv, [pv])
            lin = lin + xr
            row = lanes * NUM_FIELDS + i
            colb = (xr & 7) * LATENT_DIM
            new_accs = []
            for d in range(LATENT_DIM):
                w = plsc.load_gather(buf_ref, [row, colb + d])
                q = q + w * w
                new_accs.append(accs[d] + w)
            return (q, lin) + tuple(new_accs)

        init = (zero_f, jnp.zeros((16,), jnp.int32)) + tuple(
            zero_f for _ in range(LATENT_DIM)
        )
        res = lax.fori_loop(0, NUM_FIELDS, field_body, init)
        q, lin = res[0], res[1]
        s2 = zero_f
        for d in range(LATENT_DIM):
            s2 = s2 + res[2 + d] * res[2 + d]
        out = 0.5 * (s2 - q) + lin.astype(jnp.float32)
        out_v[pl.ds(k * CHUNK_S, CHUNK_S)] = out

    fire(0, buf.at[0], sem_a)

    def body(k2, _):
        ke = 2 * k2
        fire(ke + 1, buf.at[1], sem_b)
        drain(buf.at[0], sem_a)
        consume(ke, buf.at[0])
        fire(ke + 2, buf.at[0], sem_a)
        drain(buf.at[1], sem_b)
        consume(ke + 1, buf.at[1])
        return _

    lax.fori_loop(0, N_CHUNKS // 2 - 1, body, 0)
    fire(N_CHUNKS - 1, buf.at[1], sem_b)
    drain(buf.at[0], sem_a)
    consume(N_CHUNKS - 2, buf.at[0])
    drain(buf.at[1], sem_b)
    consume(N_CHUNKS - 1, buf.at[1])

    pltpu.sync_copy(out_v, out_hbm.at[pl.ds(wid * SAMPLES_PER_W, SAMPLES_PER_W)])


def kernel(x, field_indices, emb):
    del field_indices
    embt = jnp.transpose(emb, (0, 2, 1))
    tail = emb[:, NUM_FEATURES - TAIL_FEAT :, :].reshape(NUM_FIELDS, 4, 128)
    tail = jnp.concatenate(
        [tail, jnp.zeros((NUM_FIELDS, 4, 128), jnp.float32)], axis=1
    )
    packed = _pack_sc(embt, tail)
    u = (
        x // 8 + jnp.arange(NUM_FIELDS, dtype=jnp.int32) * F_ROWS
    ).reshape(-1)
    return _ffm_sc(packed, x.reshape(-1), u)


# speedup vs baseline: 1.1591x; 1.1591x over previous
import functools

import jax
import jax.numpy as jnp
from jax import lax
from jax.experimental import pallas as pl
from jax.experimental.pallas import tpu as pltpu
from jax.experimental.pallas import tpu_sc as plsc

NUM_FIELDS = 26
NUM_FEATURES = 100000
LATENT_DIM = 16
BATCH = 4096

NUM_WORKERS = 32
CHUNK_FEAT = 128
CHUNKS_PER_FIELD = -(-NUM_FEATURES // CHUNK_FEAT)
TAIL_FEAT = NUM_FEATURES - (CHUNKS_PER_FIELD - 1) * CHUNK_FEAT
TOTAL_CHUNKS = NUM_FIELDS * CHUNKS_PER_FIELD
SLAB_W = CHUNK_FEAT + 1
F_ROWS = CHUNKS_PER_FIELD * 16
NUM_UNITS = NUM_FIELDS * F_ROWS

SAMPLES_PER_W = BATCH // NUM_WORKERS
PAIRS_PER_W = SAMPLES_PER_W * NUM_FIELDS
CHUNK_S = 16
CHUNK_P = CHUNK_S * NUM_FIELDS
N_CHUNKS = SAMPLES_PER_W // CHUNK_S
IDX_DMA = 104


@functools.partial(
    pl.kernel,
    out_type=jax.ShapeDtypeStruct((NUM_UNITS, 128), jnp.float32),
    mesh=plsc.VectorSubcoreMesh(core_axis_name="c", subcore_axis_name="s"),
    compiler_params=pltpu.CompilerParams(needs_layout_passes=False),
    scratch_types=[
        pltpu.VMEM((16, SLAB_W), jnp.float32),
        pltpu.VMEM((16, SLAB_W), jnp.float32),
        pltpu.VMEM((16, 128), jnp.float32),
        pltpu.VMEM((16, 128), jnp.float32),
        pltpu.SemaphoreType.DMA,
        pltpu.SemaphoreType.DMA,
        pltpu.SemaphoreType.DMA,
        pltpu.SemaphoreType.DMA,
    ],
)
def _pack_sc(embt_hbm, tail_hbm, packed_hbm, sl_a, sl_b, ob_a, ob_b,
             sra, srb, swa, swb):
    wid = lax.axis_index("s") * 2 + lax.axis_index("c")
    lo = wid * TOTAL_CHUNKS // NUM_WORKERS
    hi = (wid + 1) * TOTAL_CHUNKS // NUM_WORKERS

    lanes = lax.iota(jnp.int32, 16)

    def fetch(cid, slab, sem):
        @pl.when(cid < hi)
        def _():
            i = cid // CHUNKS_PER_FIELD
            cb = cid % CHUNKS_PER_FIELD

            @pl.when(cb < CHUNKS_PER_FIELD - 1)
            def _():
                pltpu.async_copy(
                    embt_hbm.at[i, :, pl.ds(cb * CHUNK_FEAT, CHUNK_FEAT)],
                    slab.at[:, pl.ds(0, CHUNK_FEAT)],
                    sem,
                )

    def wait_fetch(cid, slab, sem):
        @pl.when((cid < hi) & (cid % CHUNKS_PER_FIELD < CHUNKS_PER_FIELD - 1))
        def _():
            pltpu.make_async_copy(embt_hbm.at[0, :, pl.ds(0, CHUNK_FEAT)],
                                  slab.at[:, pl.ds(0, CHUNK_FEAT)], sem).wait()

    def process(cid, slab, outb, semw):
        @pl.when(cid < hi)
        def _():
            i = cid // CHUNKS_PER_FIELD
            cb = cid % CHUNKS_PER_FIELD

            @pl.when(cid >= lo + 2)
            def _():
                pltpu.make_async_copy(
                    outb, packed_hbm.at[pl.ds(0, 16), :], semw
                ).wait()

            @pl.when(cb < CHUNKS_PER_FIELD - 1)
            def _():
                def tr_body(r, _):
                    for j in range(8):
                        col = jnp.full((16,), r * 8 + j, jnp.int32)
                        v = plsc.load_gather(slab, [lanes, col])
                        outb[r, pl.ds(j * 16, 16)] = v
                    return _

                lax.fori_loop(0, 16, tr_body, 0)

            @pl.when(cb == CHUNKS_PER_FIELD - 1)
            def _():
                pltpu.sync_copy(tail_hbm.at[i], outb.at[pl.ds(0, 8), :])

            pltpu.async_copy(
                outb,
                packed_hbm.at[pl.ds(i * F_ROWS + cb * 16, 16), :],
                semw,
            )

    fetch(lo, sl_a, sra)
    n = hi - lo

    def body(t, _):
        ce = lo + 2 * t
        fetch(ce + 1, sl_b, srb)
        wait_fetch(ce, sl_a, sra)
        process(ce, sl_a, ob_a, swa)
        fetch(ce + 2, sl_a, sra)
        wait_fetch(ce + 1, sl_b, srb)
        process(ce + 1, sl_b, ob_b, swb)
        return _

    lax.fori_loop(0, (n + 1) // 2, body, 0)
    pltpu.make_async_copy(ob_a, packed_hbm.at[pl.ds(0, 16), :], swa).wait()
    pltpu.make_async_copy(ob_b, packed_hbm.at[pl.ds(0, 16), :], swb).wait()


@functools.partial(
    pl.kernel,
    out_type=jax.ShapeDtypeStruct((BATCH,), jnp.float32),
    mesh=plsc.VectorSubcoreMesh(core_axis_name="c", subcore_axis_name="s"),
    compiler_params=pltpu.CompilerParams(needs_layout_passes=False),
    scratch_types=[
        pltpu.VMEM((PAIRS_PER_W,), jnp.int32),
        pltpu.VMEM((PAIRS_PER_W,), jnp.int32),
        pltpu.VMEM((2, CHUNK_P, 128), jnp.float32),
        pltpu.VMEM((SAMPLES_PER_W,), jnp.float32),
        pltpu.SemaphoreType.DMA,
        pltpu.SemaphoreType.DMA,
    ],
)
def _ffm_sc(table_hbm, x_hbm, u_hbm, out_hbm, xv, uv, buf, out_v, sem_a, sem_b):
    wid = lax.axis_index("s") * 2 + lax.axis_index("c")
    base = wid * PAIRS_PER_W

    pltpu.sync_copy(x_hbm.at[pl.ds(base, PAIRS_PER_W)], xv)
    pltpu.sync_copy(u_hbm.at[pl.ds(base, PAIRS_PER_W)], uv)

    lanes = lax.iota(jnp.int32, 16)
    zero_f = jnp.zeros((16,), jnp.float32)

    def fire(k, buf_ref, sem):
        for q in range(CHUNK_P // IDX_DMA):
            pltpu.async_copy(
                table_hbm.at[uv.at[pl.ds(k * CHUNK_P + q * IDX_DMA, IDX_DMA)]],
                buf_ref.at[pl.ds(q * IDX_DMA, IDX_DMA), :],
                sem,
            )

    def drain(buf_ref, sem):
        pltpu.make_async_copy(
            table_hbm.at[pl.ds(0, CHUNK_P), :], buf_ref, sem
        ).wait()

    def consume(k, buf_ref):
        p0 = k * CHUNK_P

        def field_body(i, carry):
            q, lin = carry[0], carry[1]
            accs = carry[2:]
            pv = p0 + lanes * NUM_FIELDS + i
            xr = plsc.load_gather(xv, [pv])
            lin = lin + xr
            row = lanes * NUM_FIELDS + i
            colb = (xr & 7) * LATENT_DIM
            new_accs = []
            for d in range(LATENT_DIM):
                w = plsc.load_gather(buf_ref, [row, colb + d])
                q = q + w * w
                new_accs.append(accs[d] + w)
            return (q, lin) + tuple(new_accs)

        init = (zero_f, jnp.zeros((16,), jnp.int32)) + tuple(
            zero_f for _ in range(LATENT_DIM)
        )
        res = lax.fori_loop(0, NUM_FIELDS, field_body, init)
        q, lin = res[0], res[1]
        s2 = zero_f
        for d in range(LATENT_DIM):
            s2 = s2 + res[2 + d] * res[2 + d]
        out = 0.5 * (s2 - q) + lin.astype(jnp.float32)
        out_v[pl.ds(k * CHUNK_S, CHUNK_S)] = out

    fire(0, buf.at[0], sem_a)

    def body(k2, _):
        ke = 2 * k2
        fire(ke + 1, buf.at[1], sem_b)
        drain(buf.at[0], sem_a)
        consume(ke, buf.at[0])
        fire(ke + 2, buf.at[0], sem_a)
        drain(buf.at[1], sem_b)
        consume(ke + 1, buf.at[1])
        return _

    lax.fori_loop(0, N_CHUNKS // 2 - 1, body, 0)
    fire(N_CHUNKS - 1, buf.at[1], sem_b)
    drain(buf.at[0], sem_a)
    consume(N_CHUNKS - 2, buf.at[0])
    drain(buf.at[1], sem_b)
    consume(N_CHUNKS - 1, buf.at[1])

    pltpu.sync_copy(out_v, out_hbm.at[pl.ds(wid * SAMPLES_PER_W, SAMPLES_PER_W)])


def kernel(x, field_indices, emb):
    del field_indices
    embt = jnp.transpose(emb, (0, 2, 1))
    tail = emb[:, NUM_FEATURES - TAIL_FEAT :, :].reshape(NUM_FIELDS, 4, 128)
    tail = jnp.concatenate(
        [tail, jnp.zeros((NUM_FIELDS, 4, 128), jnp.float32)], axis=1
    )
    packed = _pack_sc(embt, tail)
    u = (
        x // 8 + jnp.arange(NUM_FIELDS, dtype=jnp.int32) * F_ROWS
    ).reshape(-1)
    return _ffm_sc(packed, x.reshape(-1), u)
